# baseline (speedup 1.0000x reference)
import functools
import numpy as np
import jax
import jax.numpy as jnp
from jax import lax
from jax.experimental import pallas as pl
from jax.experimental.pallas import tpu as pltpu
from jax.experimental.pallas import tpu_sc as plsc

Z = 32
KC = 1024


def _argmax_body(x_ref, locs_ref, logits_ref, idx_ref):
    x = x_ref[...]
    locs = locs_ref[...]
    logits = logits_ref[...]
    scores = lax.dot_general(
        x, locs, (((1,), (1,)), ((), ())),
        preferred_element_type=jnp.float32)
    m2 = jnp.sum(locs * locs, axis=1)
    bias = (-0.5) * m2 + float(Z) * logits[0]
    scores = scores + bias[None, :]
    idx = jnp.argmax(scores, axis=-1).astype(jnp.int32)
    idx_ref[0, 0, :] = idx


def _compute_idx(xf, locs, logits, block_t):
    n = xf.shape[0]
    nb = n // block_t
    idx3 = pl.pallas_call(
        _argmax_body,
        grid=(nb,),
        in_specs=[
            pl.BlockSpec((block_t, Z), lambda i: (i, 0)),
            pl.BlockSpec((KC, Z), lambda i: (0, 0)),
            pl.BlockSpec((1, KC), lambda i: (0, 0)),
        ],
        out_specs=pl.BlockSpec((1, 1, block_t), lambda i: (i, 0, 0)),
        out_shape=jax.ShapeDtypeStruct((nb, 1, block_t), jnp.int32),
    )(xf, locs, logits.reshape(1, KC))
    return idx3.reshape(n)


def _make_sc_gather(b_total, d):
    info = plsc.get_sparse_core_info()
    nc, ns = info.num_cores, info.num_subcores
    nw = nc * ns
    assert b_total % (8 * nw) == 0
    b_per_w = b_total // nw
    mesh = plsc.VectorSubcoreMesh(core_axis_name="c", subcore_axis_name="s")

    @functools.partial(
        pl.kernel,
        mesh=mesh,
        out_type=jax.ShapeDtypeStruct((b_total, d), jnp.float32),
        scratch_types=[
            pltpu.VMEM((b_per_w,), jnp.int32),
            pltpu.VMEM((b_per_w, d), jnp.float32),
            pltpu.SemaphoreType.DMA,
        ],
        compiler_params=pltpu.CompilerParams(use_tc_tiling_on_sc=False),
    )
    def gather_kernel(table_hbm, idx_hbm, out_hbm, idx_v, rows_v, sem):
        wid = lax.axis_index("s") * nc + lax.axis_index("c")
        base = wid * b_per_w
        pltpu.sync_copy(idx_hbm.at[pl.ds(base, b_per_w)], idx_v)
        pltpu.async_copy(table_hbm.at[idx_v], rows_v, sem).wait()
        pltpu.sync_copy(rows_v, out_hbm.at[pl.ds(base, b_per_w)])

    return gather_kernel


def kernel(x, locs, logits):
    b, hw, zd = x.shape
    xf = x.reshape(b * hw, zd)
    idx = _compute_idx(xf, locs, logits, block_t=2048)
    out = _make_sc_gather(b * hw, zd)(locs, idx)
    return out.reshape(b, hw, zd)


# speedup vs baseline: 1.6752x; 1.6752x over previous
import functools
import numpy as np
import jax
import jax.numpy as jnp
from jax import lax
from jax.experimental import pallas as pl
from jax.experimental.pallas import tpu as pltpu
from jax.experimental.pallas import tpu_sc as plsc

Z = 32
KC = 1024


def _argmax_body(x_ref, locs_ref, logits_ref, idx_ref):
    x = x_ref[...]
    locs = locs_ref[...]
    logits = logits_ref[...]
    scores = lax.dot_general(
        x, locs, (((1,), (1,)), ((), ())),
        preferred_element_type=jnp.float32)
    m2 = jnp.sum(locs * locs, axis=1)
    bias = (-0.5) * m2 + float(Z) * logits[0]
    scores = scores + bias[None, :]
    idx = jnp.argmax(scores, axis=-1).astype(jnp.int32)
    idx_ref[0, 0, :] = idx


def _compute_idx(xf, locs, logits, block_t):
    n = xf.shape[0]
    nb = n // block_t
    idx3 = pl.pallas_call(
        _argmax_body,
        grid=(nb,),
        in_specs=[
            pl.BlockSpec((block_t, Z), lambda i: (i, 0)),
            pl.BlockSpec((KC, Z), lambda i: (0, 0)),
            pl.BlockSpec((1, KC), lambda i: (0, 0)),
        ],
        out_specs=pl.BlockSpec((1, 1, block_t), lambda i: (i, 0, 0)),
        out_shape=jax.ShapeDtypeStruct((nb, 1, block_t), jnp.int32),
    )(xf, locs, logits.reshape(1, KC))
    return idx3.reshape(n)


def _make_sc_gather(b_total, d):
    info = plsc.get_sparse_core_info()
    nc, ns = info.num_cores, info.num_subcores
    nw = nc * ns
    assert b_total % (8 * nw) == 0
    b_per_w = b_total // nw
    mesh = plsc.VectorSubcoreMesh(core_axis_name="c", subcore_axis_name="s")

    @functools.partial(
        pl.kernel,
        mesh=mesh,
        out_type=jax.ShapeDtypeStruct((b_total, d), jnp.float32),
        scratch_types=[
            pltpu.VMEM((b_per_w,), jnp.int32),
            pltpu.VMEM((b_per_w, d), jnp.float32),
            pltpu.VMEM_SHARED((KC, d), jnp.float32),
            pltpu.SemaphoreType.DMA,
        ],
        compiler_params=pltpu.CompilerParams(use_tc_tiling_on_sc=False),
    )
    def gather_kernel(table_hbm, idx_hbm, out_hbm, idx_v, rows_v, table_sh, sem):
        cid = lax.axis_index("c")
        sid = lax.axis_index("s")
        wid = sid * nc + cid
        base = wid * b_per_w

        @pl.when(sid == 0)
        def _():
            pltpu.sync_copy(table_hbm, table_sh)

        pltpu.sync_copy(idx_hbm.at[pl.ds(base, b_per_w)], idx_v)
        plsc.subcore_barrier()
        pltpu.async_copy(table_sh.at[idx_v], rows_v, sem).wait()
        pltpu.sync_copy(rows_v, out_hbm.at[pl.ds(base, b_per_w)])

    return gather_kernel


def kernel(x, locs, logits):
    b, hw, zd = x.shape
    xf = x.reshape(b * hw, zd)
    idx = _compute_idx(xf, locs, logits, block_t=2048)
    out = _make_sc_gather(b * hw, zd)(locs, idx)
    return out.reshape(b, hw, zd)
